# baseline (speedup 1.0000x reference)
import functools

import jax
import jax.numpy as jnp
from jax import lax
from jax.experimental import pallas as pl
from jax.experimental.pallas import tpu as pltpu
from jax.experimental.pallas import tpu_sc as plsc

B = 16384
D = 64
CHUNK = 128
NC, NS = 2, 16
NW = NC * NS
B_PER_W = B // NW
K_PER_W = B_PER_W // CHUNK
DMA_BLOCK = 32
NBLK = B_PER_W // DMA_BLOCK


def _sc_gather_body(ptab_hbm, atab_hbm, pidx_hbm, aidx_hbm,
                    pe_hbm, ae_hbm,
                    pidx_v, aidx_v, rows_v, ae_v0, ae_v1, sem, asem):
    wid = lax.axis_index("s") * NC + lax.axis_index("c")
    rowbase = wid * B_PER_W
    pltpu.sync_copy(pidx_hbm.at[pl.ds(wid, 1)], pidx_v)
    pltpu.sync_copy(aidx_hbm.at[pl.ds(wid * K_PER_W, K_PER_W)], aidx_v)

    ae_bufs = [ae_v0, ae_v1]
    a_copies = [pltpu.async_copy(atab_hbm.at[aidx_v.at[0]], ae_v0, asem)]

    prev = None
    for b in range(NBLK):
        cur = []
        for g in range(DMA_BLOCK // 16):
            v = pidx_v[0, pl.ds(b * DMA_BLOCK + g * 16, 16)]
            for l in range(16):
                i = b * DMA_BLOCK + g * 16 + l
                cur.append(pltpu.async_copy(
                    ptab_hbm.at[pl.ds(v[l], 1)], rows_v.at[pl.ds(i, 1)], sem))
        if prev is not None:
            for c in prev:
                c.wait()
        prev = cur
        j = b // (NBLK // K_PER_W)
        if b % (NBLK // K_PER_W) == 1 and j + 1 < K_PER_W:
            a_copies.append(pltpu.async_copy(
                atab_hbm.at[aidx_v.at[j + 1]], ae_bufs[(j + 1) % 2], asem))
        if b % (NBLK // K_PER_W) == 2:
            a_copies[j].wait()
            pltpu.sync_copy(ae_bufs[j % 2],
                            ae_hbm.at[pl.ds(rowbase + j * CHUNK, CHUNK)])
    for c in prev:
        c.wait()
    pltpu.sync_copy(rows_v, pe_hbm.at[pl.ds(rowbase, B_PER_W)])


@jax.jit
def _sc_gather(product_table_u32, action_type_table, product_ids, action_types):
    atab_p = jnp.pad(action_type_table, ((0, 0), (0, 64)))
    pidx = product_ids.reshape(NW, B_PER_W)
    aidx = action_types.reshape(B // CHUNK, CHUNK)
    mesh = plsc.VectorSubcoreMesh(core_axis_name="c", subcore_axis_name="s")
    out_t = (jax.ShapeDtypeStruct((B, D // 2), jnp.uint32),
             jax.ShapeDtypeStruct((B, 128), jnp.float32))
    fn = pl.kernel(
        _sc_gather_body,
        mesh=mesh,
        out_type=out_t,
        scratch_types=[
            pltpu.VMEM((1, B_PER_W), jnp.int32),
            pltpu.VMEM((K_PER_W, CHUNK), jnp.int32),
            pltpu.VMEM((B_PER_W, D // 2), jnp.uint32),
            pltpu.VMEM((CHUNK, 128), jnp.float32),
            pltpu.VMEM((CHUNK, 128), jnp.float32),
            pltpu.SemaphoreType.DMA,
            pltpu.SemaphoreType.DMA,
        ],
    )
    return fn(product_table_u32, atab_p, pidx, aidx)


def _dense_body(ae_ref, pe_ref, tf_ref, cf_ref,
                tw_ref, tb_ref, cw_ref, cb_ref,
                wa_ref, wp0_ref, wp1_ref, wt_ref, wc_ref, fb_ref, out_ref):
    f32 = jnp.float32
    cdims = (((1,), (1,)), ((), ()))
    t_emb = lax.dot_general(tf_ref[...], tw_ref[...], cdims,
                            preferred_element_type=f32) + tb_ref[...]
    c_emb = lax.dot_general(cf_ref[...], cw_ref[...], cdims,
                            preferred_element_type=f32) + cb_ref[...]
    mm = (((1,), (0,)), ((), ()))
    acc = lax.dot_general(ae_ref[...], wa_ref[...], mm, preferred_element_type=f32)
    pe_u = pe_ref[...]
    pe_even = lax.bitcast_convert_type(pe_u << 16, f32)
    pe_odd = lax.bitcast_convert_type(pe_u & jnp.uint32(0xFFFF0000), f32)
    acc += lax.dot_general(pe_even, wp0_ref[...], mm, preferred_element_type=f32)
    acc += lax.dot_general(pe_odd, wp1_ref[...], mm, preferred_element_type=f32)
    acc += lax.dot_general(t_emb, wt_ref[...], mm, preferred_element_type=f32)
    acc += lax.dot_general(c_emb, wc_ref[...], mm, preferred_element_type=f32)
    out_ref[...] = jnp.maximum(acc + fb_ref[...], 0.0)


@functools.partial(jax.jit, static_argnames=("blk",))
def _tc_dense(action_emb, product_emb, temporal_features, context_features,
              temporal_W, temporal_b, context_W, context_b,
              wa, wp0, wp1, wt, wc, fb, blk=2048):
    grid = (B // blk,)
    row_spec = lambda d: pl.BlockSpec((blk, d), lambda i: (i, 0))
    full = lambda a: pl.BlockSpec(a.shape, lambda i: (0,) * a.ndim)
    return pl.pallas_call(
        _dense_body,
        grid=grid,
        in_specs=[
            row_spec(128), row_spec(D // 2), row_spec(5), row_spec(10),
            full(temporal_W), full(temporal_b), full(context_W), full(context_b),
            full(wa), full(wp0), full(wp1), full(wt), full(wc), full(fb),
        ],
        out_specs=pl.BlockSpec((blk, 128), lambda i: (i, 0)),
        out_shape=jax.ShapeDtypeStruct((B, 128), jnp.float32),
    )(action_emb, product_emb, temporal_features, context_features,
      temporal_W, temporal_b, context_W, context_b, wa, wp0, wp1, wt, wc, fb)


def kernel(action_types, product_ids, temporal_features, context_features,
           action_type_table, product_table,
           temporal_W, temporal_b, context_W, context_b,
           fusion_W, fusion_b):
    tab_u = lax.bitcast_convert_type(product_table, jnp.uint32)
    tab3 = tab_u.reshape(1000000, D // 2, 2)
    packed = (tab3[:, :, 0] >> 16) | (tab3[:, :, 1] & jnp.uint32(0xFFFF0000))
    pe, ae_w = _sc_gather(packed, action_type_table,
                          product_ids, action_types)
    wa_p = jnp.concatenate(
        [fusion_W[:, 0:64].T, jnp.zeros((64, 128), jnp.float32)], axis=0)
    wp = fusion_W[:, 64:128].T
    wt = fusion_W[:, 128:160].T
    wc = fusion_W[:, 160:192].T
    return _tc_dense(ae_w, pe, temporal_features, context_features,
                     temporal_W, temporal_b.reshape(1, 32),
                     context_W, context_b.reshape(1, 32),
                     wa_p, wp[0::2], wp[1::2], wt, wc,
                     fusion_b.reshape(1, 128))


# speedup vs baseline: 5.7745x; 5.7745x over previous
import functools

import jax
import jax.numpy as jnp
from jax import lax
from jax.experimental import pallas as pl
from jax.experimental.pallas import tpu as pltpu
from jax.experimental.pallas import tpu_sc as plsc

B = 16384
D = 64
CHUNK = 128
NC, NS = 2, 16
NW = NC * NS
B_PER_W = B // NW
K_PER_W = B_PER_W // CHUNK
DMA_BLOCK = 32
NBLK = B_PER_W // DMA_BLOCK


def _sc_gather_body(ptab_hbm, atab_hbm, pidx_hbm, aidx_hbm,
                    pe_hbm, ae_hbm,
                    pidx_v, aidx_v, rows_v, ae_v0, ae_v1, sem, asem):
    wid = lax.axis_index("s") * NC + lax.axis_index("c")
    rowbase = wid * B_PER_W
    pltpu.sync_copy(pidx_hbm.at[pl.ds(wid, 1)], pidx_v)
    pltpu.sync_copy(aidx_hbm.at[pl.ds(wid * K_PER_W, K_PER_W)], aidx_v)

    ae_bufs = [ae_v0, ae_v1]
    a_copies = [pltpu.async_copy(atab_hbm.at[aidx_v.at[0]], ae_v0, asem)]

    prev = None
    for b in range(NBLK):
        cur = []
        for g in range(DMA_BLOCK // 16):
            v = pidx_v[0, pl.ds(b * DMA_BLOCK + g * 16, 16)]
            for l in range(16):
                i = b * DMA_BLOCK + g * 16 + l
                cur.append(pltpu.async_copy(
                    ptab_hbm.at[pl.ds(v[l], 1)], rows_v.at[pl.ds(i, 1)], sem))
        if prev is not None:
            for c in prev:
                c.wait()
        prev = cur
        j = b // (NBLK // K_PER_W)
        if b % (NBLK // K_PER_W) == 1 and j + 1 < K_PER_W:
            a_copies.append(pltpu.async_copy(
                atab_hbm.at[aidx_v.at[j + 1]], ae_bufs[(j + 1) % 2], asem))
        if b % (NBLK // K_PER_W) == 2:
            a_copies[j].wait()
            pltpu.sync_copy(ae_bufs[j % 2],
                            ae_hbm.at[pl.ds(rowbase + j * CHUNK, CHUNK)])
    for c in prev:
        c.wait()
    pltpu.sync_copy(rows_v, pe_hbm.at[pl.ds(rowbase, B_PER_W)])


@jax.jit
def _sc_gather(product_table, action_type_table, product_ids, action_types):
    atab_p = jnp.pad(action_type_table, ((0, 0), (0, 64)))
    pidx = product_ids.reshape(NW, B_PER_W)
    aidx = action_types.reshape(B // CHUNK, CHUNK)
    mesh = plsc.VectorSubcoreMesh(core_axis_name="c", subcore_axis_name="s")
    out_t = (jax.ShapeDtypeStruct((B, D), jnp.float32),
             jax.ShapeDtypeStruct((B, 128), jnp.float32))
    fn = pl.kernel(
        _sc_gather_body,
        mesh=mesh,
        out_type=out_t,
        scratch_types=[
            pltpu.VMEM((1, B_PER_W), jnp.int32),
            pltpu.VMEM((K_PER_W, CHUNK), jnp.int32),
            pltpu.VMEM((B_PER_W, D), jnp.float32),
            pltpu.VMEM((CHUNK, 128), jnp.float32),
            pltpu.VMEM((CHUNK, 128), jnp.float32),
            pltpu.SemaphoreType.DMA,
            pltpu.SemaphoreType.DMA,
        ],
    )
    return fn(product_table, atab_p, pidx, aidx)


def _dense_body(ae_ref, pe_ref, tf_ref, cf_ref,
                tw_ref, tb_ref, cw_ref, cb_ref,
                wa_ref, wp_ref, wt_ref, wc_ref, fb_ref, out_ref):
    f32 = jnp.float32
    cdims = (((1,), (1,)), ((), ()))
    t_emb = lax.dot_general(tf_ref[...], tw_ref[...], cdims,
                            preferred_element_type=f32) + tb_ref[...]
    c_emb = lax.dot_general(cf_ref[...], cw_ref[...], cdims,
                            preferred_element_type=f32) + cb_ref[...]
    mm = (((1,), (0,)), ((), ()))
    acc = lax.dot_general(ae_ref[...], wa_ref[...], mm, preferred_element_type=f32)
    acc += lax.dot_general(pe_ref[...], wp_ref[...], mm, preferred_element_type=f32)
    acc += lax.dot_general(t_emb, wt_ref[...], mm, preferred_element_type=f32)
    acc += lax.dot_general(c_emb, wc_ref[...], mm, preferred_element_type=f32)
    out_ref[...] = jnp.maximum(acc + fb_ref[...], 0.0)


@functools.partial(jax.jit, static_argnames=("blk",))
def _tc_dense(action_emb, product_emb, temporal_features, context_features,
              temporal_W, temporal_b, context_W, context_b,
              wa, wp, wt, wc, fb, blk=2048):
    grid = (B // blk,)
    row_spec = lambda d: pl.BlockSpec((blk, d), lambda i: (i, 0))
    full = lambda a: pl.BlockSpec(a.shape, lambda i: (0,) * a.ndim)
    return pl.pallas_call(
        _dense_body,
        grid=grid,
        in_specs=[
            row_spec(128), row_spec(D), row_spec(5), row_spec(10),
            full(temporal_W), full(temporal_b), full(context_W), full(context_b),
            full(wa), full(wp), full(wt), full(wc), full(fb),
        ],
        out_specs=pl.BlockSpec((blk, 128), lambda i: (i, 0)),
        out_shape=jax.ShapeDtypeStruct((B, 128), jnp.float32),
    )(action_emb, product_emb, temporal_features, context_features,
      temporal_W, temporal_b, context_W, context_b, wa, wp, wt, wc, fb)


def kernel(action_types, product_ids, temporal_features, context_features,
           action_type_table, product_table,
           temporal_W, temporal_b, context_W, context_b,
           fusion_W, fusion_b):
    pe, ae_w = _sc_gather(product_table, action_type_table,
                          product_ids, action_types)
    wa_p = jnp.concatenate(
        [fusion_W[:, 0:64].T, jnp.zeros((64, 128), jnp.float32)], axis=0)
    wp = fusion_W[:, 64:128].T
    wt = fusion_W[:, 128:160].T
    wc = fusion_W[:, 160:192].T
    return _tc_dense(ae_w, pe, temporal_features, context_features,
                     temporal_W, temporal_b.reshape(1, 32),
                     context_W, context_b.reshape(1, 32),
                     wa_p, wp, wt, wc, fusion_b.reshape(1, 128))


# speedup vs baseline: 6.4367x; 1.1147x over previous
import functools

import jax
import jax.numpy as jnp
from jax import lax
from jax.experimental import pallas as pl
from jax.experimental.pallas import tpu as pltpu
from jax.experimental.pallas import tpu_sc as plsc

B = 16384
D = 64
CHUNK = 128
NC, NS = 2, 16
NW = NC * NS
B_PER_W = B // NW
K_PER_W = B_PER_W // CHUNK
DMA_BLOCK = 32
NBLK = B_PER_W // DMA_BLOCK


def _sc_gather_body(ptab_hbm, pidx_hbm, pe_hbm, pidx_v, rows_v, sem):
    wid = lax.axis_index("s") * NC + lax.axis_index("c")
    rowbase = wid * B_PER_W
    pltpu.sync_copy(pidx_hbm.at[pl.ds(wid, 1)], pidx_v)

    prev = None
    for b in range(NBLK):
        cur = []
        for g in range(DMA_BLOCK // 16):
            v = pidx_v[0, pl.ds(b * DMA_BLOCK + g * 16, 16)]
            for l in range(16):
                i = b * DMA_BLOCK + g * 16 + l
                cur.append(pltpu.async_copy(
                    ptab_hbm.at[pl.ds(v[l], 1)], rows_v.at[pl.ds(i, 1)], sem))
        if prev is not None:
            for c in prev:
                c.wait()
        prev = cur
    for c in prev:
        c.wait()
    pltpu.sync_copy(rows_v, pe_hbm.at[pl.ds(rowbase, B_PER_W)])


@jax.jit
def _sc_gather(product_table, product_ids):
    pidx = product_ids.reshape(NW, B_PER_W)
    mesh = plsc.VectorSubcoreMesh(core_axis_name="c", subcore_axis_name="s")
    fn = pl.kernel(
        _sc_gather_body,
        mesh=mesh,
        out_type=jax.ShapeDtypeStruct((B, D), jnp.float32),
        scratch_types=[
            pltpu.VMEM((1, B_PER_W), jnp.int32),
            pltpu.VMEM((B_PER_W, D), jnp.float32),
            pltpu.SemaphoreType.DMA,
        ],
    )
    return fn(product_table, pidx)


def _dense_body(ats_ref, pe_ref, tf_ref, cf_ref,
                atab_ref, tw_ref, tb_ref, cw_ref, cb_ref,
                wa_ref, wp_ref, wt_ref, wc_ref, fb_ref, out_ref):
    f32 = jnp.float32
    cdims = (((1,), (1,)), ((), ()))
    t_emb = lax.dot_general(tf_ref[...], tw_ref[...], cdims,
                            preferred_element_type=f32) + tb_ref[...]
    c_emb = lax.dot_general(cf_ref[...], cw_ref[...], cdims,
                            preferred_element_type=f32) + cb_ref[...]
    mm = (((1,), (0,)), ((), ()))
    ats_lane = ats_ref[0, 0, :].astype(f32).reshape(1, -1)
    ats_col = lax.dot_general(ats_lane, jnp.ones((1, 1), f32),
                              (((0,), (0,)), ((), ())),
                              preferred_element_type=f32)
    iota20 = lax.broadcasted_iota(jnp.int32, (1, 20), 1).astype(f32)
    oh = (ats_col == iota20).astype(f32)
    wa_eff = lax.dot_general(atab_ref[...], wa_ref[...], mm,
                             preferred_element_type=f32)
    acc = lax.dot_general(oh, wa_eff, mm, preferred_element_type=f32)
    acc += lax.dot_general(pe_ref[...], wp_ref[...], mm, preferred_element_type=f32)
    acc += lax.dot_general(t_emb, wt_ref[...], mm, preferred_element_type=f32)
    acc += lax.dot_general(c_emb, wc_ref[...], mm, preferred_element_type=f32)
    out_ref[...] = jnp.maximum(acc + fb_ref[...], 0.0)


@functools.partial(jax.jit, static_argnames=("blk",))
def _tc_dense(action_types3, product_emb, temporal_features, context_features,
              action_type_table, temporal_W, temporal_b, context_W, context_b,
              wa, wp, wt, wc, fb, blk=2048):
    grid = (B // blk,)
    row_spec = lambda d: pl.BlockSpec((blk, d), lambda i: (i, 0))
    full = lambda a: pl.BlockSpec(a.shape, lambda i: (0,) * a.ndim)
    return pl.pallas_call(
        _dense_body,
        grid=grid,
        in_specs=[
            pl.BlockSpec((1, 1, blk), lambda i: (i, 0, 0)),
            row_spec(D), row_spec(5), row_spec(10),
            full(action_type_table),
            full(temporal_W), full(temporal_b), full(context_W), full(context_b),
            full(wa), full(wp), full(wt), full(wc), full(fb),
        ],
        out_specs=pl.BlockSpec((blk, 128), lambda i: (i, 0)),
        out_shape=jax.ShapeDtypeStruct((B, 128), jnp.float32),
    )(action_types3, product_emb, temporal_features, context_features,
      action_type_table, temporal_W, temporal_b, context_W, context_b,
      wa, wp, wt, wc, fb)


def kernel(action_types, product_ids, temporal_features, context_features,
           action_type_table, product_table,
           temporal_W, temporal_b, context_W, context_b,
           fusion_W, fusion_b):
    pe = _sc_gather(product_table, product_ids)
    wa = fusion_W[:, 0:64].T
    wp = fusion_W[:, 64:128].T
    wt = fusion_W[:, 128:160].T
    wc = fusion_W[:, 160:192].T
    ats3 = action_types.reshape(B // 2048, 1, 2048)
    return _tc_dense(ats3, pe, temporal_features, context_features,
                     action_type_table,
                     temporal_W, temporal_b.reshape(1, 32),
                     context_W, context_b.reshape(1, 32),
                     wa, wp, wt, wc, fusion_b.reshape(1, 128))


# speedup vs baseline: 6.4703x; 1.0052x over previous
import functools

import jax
import jax.numpy as jnp
from jax import lax
from jax.experimental import pallas as pl
from jax.experimental.pallas import tpu as pltpu
from jax.experimental.pallas import tpu_sc as plsc

B = 16384
D = 64
NC, NS = 2, 16
NW = NC * NS
B_PER_W = B // NW
DMA_BLOCK = 32
NBLK = B_PER_W // DMA_BLOCK


def _sc_gather_body(ptab_hbm, pidx_hbm, pe_hbm, pidx_v, rows_v, sem):
    wid = lax.axis_index("s") * NC + lax.axis_index("c")
    rowbase = wid * B_PER_W
    pltpu.sync_copy(pidx_hbm.at[pl.ds(wid, 1)], pidx_v)

    prev = None
    for b in range(NBLK):
        cur = []
        for g in range(DMA_BLOCK // 16):
            v = pidx_v[0, pl.ds(b * DMA_BLOCK + g * 16, 16)]
            for l in range(16):
                i = b * DMA_BLOCK + g * 16 + l
                cur.append(pltpu.async_copy(
                    ptab_hbm.at[pl.ds(v[l], 1)], rows_v.at[pl.ds(i, 1)], sem))
        if prev is not None:
            for c in prev:
                c.wait()
        prev = cur
    for c in prev:
        c.wait()
    pltpu.sync_copy(rows_v, pe_hbm.at[pl.ds(rowbase, B_PER_W)])


@jax.jit
def _sc_gather(product_table, product_ids):
    pidx = product_ids.reshape(NW, B_PER_W)
    mesh = plsc.VectorSubcoreMesh(core_axis_name="c", subcore_axis_name="s")
    fn = pl.kernel(
        _sc_gather_body,
        mesh=mesh,
        out_type=jax.ShapeDtypeStruct((B, D), jnp.float32),
        scratch_types=[
            pltpu.VMEM((1, B_PER_W), jnp.int32),
            pltpu.VMEM((B_PER_W, D), jnp.float32),
            pltpu.SemaphoreType.DMA,
        ],
    )
    return fn(product_table, pidx)


def _dense_body(ats_ref, pe_ref, tf_ref, cf_ref,
                atab_ref, tw_ref, tb_ref, cw_ref, cb_ref,
                wa_ref, wp_ref, wt_ref, wc_ref, fb_ref, out_ref):
    f32 = jnp.float32
    cdims = (((1,), (1,)), ((), ()))
    t_emb = lax.dot_general(tf_ref[...], tw_ref[...], cdims,
                            preferred_element_type=f32) + tb_ref[...]
    c_emb = lax.dot_general(cf_ref[...], cw_ref[...], cdims,
                            preferred_element_type=f32) + cb_ref[...]
    mm = (((1,), (0,)), ((), ()))
    ats_lane = ats_ref[0, 0, :].astype(f32).reshape(1, -1)
    ats_col = lax.dot_general(ats_lane, jnp.ones((1, 1), f32),
                              (((0,), (0,)), ((), ())),
                              preferred_element_type=f32)
    iota20 = lax.broadcasted_iota(jnp.int32, (1, 20), 1).astype(f32)
    oh = (ats_col == iota20).astype(f32)
    wa_eff = lax.dot_general(atab_ref[...], wa_ref[...], mm,
                             preferred_element_type=f32)
    acc = lax.dot_general(oh, wa_eff, mm, preferred_element_type=f32)
    acc += lax.dot_general(pe_ref[...], wp_ref[...], mm, preferred_element_type=f32)
    acc += lax.dot_general(t_emb, wt_ref[...], mm, preferred_element_type=f32)
    acc += lax.dot_general(c_emb, wc_ref[...], mm, preferred_element_type=f32)
    out_ref[...] = jnp.maximum(acc + fb_ref[...], 0.0)


@functools.partial(jax.jit, static_argnames=("blk",))
def _tc_dense(action_types3, product_emb, temporal_features, context_features,
              action_type_table, temporal_W, temporal_b, context_W, context_b,
              wa, wp, wt, wc, fb, blk=2048):
    grid = (B // blk,)
    row_spec = lambda d: pl.BlockSpec((blk, d), lambda i: (i, 0))
    full = lambda a: pl.BlockSpec(a.shape, lambda i: (0,) * a.ndim)
    return pl.pallas_call(
        _dense_body,
        grid=grid,
        in_specs=[
            pl.BlockSpec((1, 1, blk), lambda i: (i, 0, 0)),
            row_spec(D), row_spec(5), row_spec(10),
            full(action_type_table),
            full(temporal_W), full(temporal_b), full(context_W), full(context_b),
            full(wa), full(wp), full(wt), full(wc), full(fb),
        ],
        out_specs=pl.BlockSpec((blk, 128), lambda i: (i, 0)),
        out_shape=jax.ShapeDtypeStruct((B, 128), jnp.float32),
    )(action_types3, product_emb, temporal_features, context_features,
      action_type_table, temporal_W, temporal_b, context_W, context_b,
      wa, wp, wt, wc, fb)


def kernel(action_types, product_ids, temporal_features, context_features,
           action_type_table, product_table,
           temporal_W, temporal_b, context_W, context_b,
           fusion_W, fusion_b):
    pe = _sc_gather(product_table, product_ids)
    wa = fusion_W[:, 0:64].T
    wp = fusion_W[:, 64:128].T
    wt = fusion_W[:, 128:160].T
    wc = fusion_W[:, 160:192].T
    ats3 = action_types.reshape(B // 2048, 1, 2048)
    return _tc_dense(ats3, pe, temporal_features, context_features,
                     action_type_table,
                     temporal_W, temporal_b.reshape(1, 32),
                     context_W, context_b.reshape(1, 32),
                     wa, wp, wt, wc, fusion_b.reshape(1, 128))
